# baseline (speedup 1.0000x reference)
import functools

import jax
import jax.numpy as jnp
from jax import lax
from jax.experimental import pallas as pl
from jax.experimental.pallas import tpu as pltpu
from jax.experimental.pallas import tpu_sc as plsc

_NC = 2
_NS = 16
_NW = _NC * _NS
_NBUF = 4
_CHUNK = 248


def kernel(x_dict, edge_index, entity_emb, rel_emb):
    del x_dict, edge_index
    n_ent, d_ent = entity_emb.shape
    rows = (n_ent // _NW) // _CHUNK * _CHUNK
    nchunks = rows // _CHUNK
    tail_base = rows * _NW
    tail = n_ent - tail_base

    mesh = plsc.VectorSubcoreMesh(core_axis_name="c", subcore_axis_name="s")

    @functools.partial(
        pl.kernel,
        mesh=mesh,
        out_type=[
            jax.ShapeDtypeStruct(entity_emb.shape, entity_emb.dtype),
            jax.ShapeDtypeStruct(rel_emb.shape, rel_emb.dtype),
        ],
        scratch_types=(
            [pltpu.VMEM((_CHUNK, d_ent), entity_emb.dtype) for _ in range(_NBUF)]
            + [pltpu.SemaphoreType.DMA for _ in range(2 * _NBUF + 1)]
        ),
    )
    def _sc_copy(ent_hbm, rel_hbm, ent_out, rel_out, *scratch):
        bufs = scratch[:_NBUF]
        isems = scratch[_NBUF:2 * _NBUF]
        osems = scratch[2 * _NBUF:3 * _NBUF]
        rsem = scratch[3 * _NBUF]
        wid = lax.axis_index("s") * _NC + lax.axis_index("c")
        base = wid * rows

        @pl.when(wid == 0)
        def _():
            pltpu.make_async_copy(rel_hbm, rel_out, rsem).start()
            if tail:
                pltpu.make_async_copy(
                    ent_hbm.at[pl.ds(tail_base, tail)],
                    ent_out.at[pl.ds(tail_base, tail)],
                    rsem,
                ).start()

        in_cps = [None] * _NBUF
        out_cps = [None] * _NBUF
        for i in range(min(_NBUF, nchunks)):
            lo = base + i * _CHUNK
            cp = pltpu.make_async_copy(
                ent_hbm.at[pl.ds(lo, _CHUNK)], bufs[i], isems[i]
            )
            cp.start()
            in_cps[i] = cp
        for i in range(nchunks):
            b = i % _NBUF
            in_cps[b].wait()
            lo = base + i * _CHUNK
            cp = pltpu.make_async_copy(
                bufs[b], ent_out.at[pl.ds(lo, _CHUNK)], osems[b]
            )
            cp.start()
            out_cps[b] = cp
            nxt = i + _NBUF
            if nxt < nchunks:
                out_cps[b].wait()
                nlo = base + nxt * _CHUNK
                ncp = pltpu.make_async_copy(
                    ent_hbm.at[pl.ds(nlo, _CHUNK)], bufs[b], isems[b]
                )
                ncp.start()
                in_cps[b] = ncp
        for cp in out_cps:
            if cp is not None:
                cp.wait()

        @pl.when(wid == 0)
        def _():
            if tail:
                pltpu.make_async_copy(
                    ent_hbm.at[pl.ds(tail_base, tail)],
                    ent_out.at[pl.ds(tail_base, tail)],
                    rsem,
                ).wait()
            pltpu.make_async_copy(rel_hbm, rel_out, rsem).wait()

    return tuple(_sc_copy(entity_emb, rel_emb))


# speedup vs baseline: 1.1273x; 1.1273x over previous
import functools

import jax
import jax.numpy as jnp
from jax import lax
from jax.experimental import pallas as pl
from jax.experimental.pallas import tpu as pltpu
from jax.experimental.pallas import tpu_sc as plsc

_NC = 2
_NS = 16
_BLK = 25000


def _copy_block(ent_ref, ent_out):
    ent_out[...] = ent_ref[...]


def _sc_rel_copy(rel_emb):
    mesh = plsc.VectorSubcoreMesh(core_axis_name="c", subcore_axis_name="s")

    @functools.partial(
        pl.kernel,
        mesh=mesh,
        out_type=jax.ShapeDtypeStruct(rel_emb.shape, rel_emb.dtype),
        scratch_types=[pltpu.SemaphoreType.DMA],
    )
    def _body(rel_in, rel_out, sem):
        wid = lax.axis_index("s") * _NC + lax.axis_index("c")

        @pl.when(wid == 0)
        def _():
            cp = pltpu.make_async_copy(rel_in, rel_out, sem)
            cp.start()
            cp.wait()

    return _body(rel_emb)


def kernel(x_dict, edge_index, entity_emb, rel_emb):
    del x_dict, edge_index
    n_ent, d_ent = entity_emb.shape
    ent = pl.pallas_call(
        _copy_block,
        grid=(n_ent // _BLK,),
        in_specs=[pl.BlockSpec((_BLK, d_ent), lambda i: (i, 0))],
        out_specs=pl.BlockSpec((_BLK, d_ent), lambda i: (i, 0)),
        out_shape=jax.ShapeDtypeStruct((n_ent, d_ent), entity_emb.dtype),
    )(entity_emb)
    rel = _sc_rel_copy(rel_emb)
    return (ent, rel)
